# Initial kernel scaffold; baseline (speedup 1.0000x reference)
import jax
import jax.numpy as jnp
from jax.experimental import pallas as pl


def kernel(user_emb, item_emb, adj_row, adj_col, adj_val):
    raise NotImplementedError("write your pallas kernel here")


# speedup vs baseline: 4.0383x; 4.0383x over previous
import functools

import jax
import jax.numpy as jnp
from jax import lax
from jax.experimental import pallas as pl
from jax.experimental.pallas import tpu as pltpu
from jax.experimental.pallas import tpu_sc as plsc

N_USERS = 50000
N_ITEMS = 50000
N_NODES = N_USERS + N_ITEMS
NNZ = 1600000
D = 32

_SUPER = 1024
_GB = 128
_NG = _SUPER // _GB
_EP = 200
_NPASS = 2


def _build(n_users, nnz, super_, gb, ep, npass):
    d = D
    ng = super_ // gb
    lanes = 16
    n_sub = 16
    qrows = n_users // npass
    nblk = qrows // ep
    nb_per_tile = -(-nblk // n_sub)
    assert qrows % ep == 0 and ep % 8 == 0
    assert super_ % (8 * lanes) == 0 and (ep * d) % lanes == 0

    mesh = plsc.VectorSubcoreMesh(
        core_axis_name="c", subcore_axis_name="s", num_cores=2, num_subcores=n_sub
    )

    def body(ego, colp, rowp, valp, bounds, u_out, i_out,
             bounds_v, colf, rowf, valf, col2d, lidx2d, rows_buf,
             acc_v, ego_v, out_v, accum, sem, sem2):
        c = lax.axis_index("c").astype(jnp.int32)
        s = lax.axis_index("s").astype(jnp.int32)
        i16 = lax.broadcasted_iota(jnp.int32, (lanes,), 0)
        zero16 = jnp.zeros((lanes,), jnp.float32)

        pltpu.sync_copy(bounds, bounds_v)
        bv = bounds_v[...]
        bnd = [jnp.int32(0)] + [bv[q] for q in range(2 * npass - 1)] + [jnp.int32(nnz)]

        coef = jnp.where(c == 0, jnp.float32(0.5), jnp.float32(1.5))

        for p in range(npass):
            lo_sc = jnp.where(c == 0, bnd[p], bnd[npass + p])
            hi_sc = jnp.where(c == 0, bnd[p + 1], bnd[npass + p + 1])
            row_base = c * n_users + p * qrows

            cnt = hi_sc - lo_sc
            per = (cnt + (n_sub - 1)) // n_sub
            lo = lo_sc + jnp.minimum(s * per, cnt)
            hi = lo_sc + jnp.minimum((s + 1) * per, cnt)
            base = (lo // 8) * 8
            n_chunks = (hi - base + (super_ - 1)) // super_

            def zvreg(t, carry):
                acc_v[t, pl.ds(0, lanes)] = zero16
                acc_v[t, pl.ds(lanes, lanes)] = zero16
                return carry
            lax.fori_loop(0, ep, zvreg, 0)

            def zcopy(i, carry):
                bid = s + i * n_sub
                @pl.when(bid < nblk)
                def _():
                    pltpu.sync_copy(acc_v, accum.at[pl.ds(bid * ep, ep)])
                return carry
            lax.fori_loop(0, nb_per_tile, zcopy, 0)
            plsc.subcore_barrier()

            def chunk(g, carry):
                start = base + g * super_
                pltpu.sync_copy(colp.at[pl.ds(start, super_)], colf)
                pltpu.sync_copy(rowp.at[pl.ds(start, super_)], rowf)
                pltpu.sync_copy(valp.at[pl.ds(start, super_)], valf)

                def prep(k, carry2):
                    for sub in range(super_ // lanes // ng):
                        off = sub * lanes
                        el = k * (super_ // ng) + off + i16
                        e = start + el
                        valid = (e >= lo) & (e < hi)
                        cidx = plsc.load_gather(colf, [el])
                        r = plsc.load_gather(rowf, [el])
                        v = plsc.load_gather(valf, [el])
                        lr = jnp.where(valid, r - row_base, 0)
                        v = jnp.where(valid, v, 0.0)
                        plsc.store_scatter(valf, [el], v)
                        col2d[k, pl.ds(off, lanes)] = cidx
                        lidx2d[k, pl.ds(off, lanes)] = lr
                    return carry2
                lax.fori_loop(0, ng, prep, 0)

                gets = [pltpu.async_copy(ego.at[col2d.at[k]],
                                         rows_buf.at[pl.ds(k * gb, gb)], sem)
                        for k in range(ng)]
                for dsc in gets:
                    dsc.wait()

                def scale(j, carry2):
                    el = j * lanes + i16
                    v16 = plsc.load_gather(valf, [el])
                    for dd in range(d):
                        d16 = jnp.full((lanes,), dd, jnp.int32)
                        gv = plsc.load_gather(rows_buf, [el, d16])
                        plsc.store_scatter(rows_buf, [el, d16], gv * v16)
                    return carry2
                lax.fori_loop(0, super_ // lanes, scale, 0)

                puts = [pltpu.async_copy(rows_buf.at[pl.ds(k * gb, gb)],
                                         accum.at[lidx2d.at[k]], sem2, add=True)
                        for k in range(ng)]
                for dsc in puts:
                    dsc.wait()
                return carry
            lax.fori_loop(0, n_chunks, chunk, 0)
            plsc.subcore_barrier()

            def ep_blk(i, carry):
                bid = s + i * n_sub

                @pl.when(bid < nblk)
                def _():
                    r0 = bid * ep
                    pltpu.sync_copy(accum.at[pl.ds(r0, ep)], acc_v)
                    pltpu.sync_copy(ego.at[pl.ds(row_base + r0, ep)], ego_v)

                    def comp(t, carry2):
                        a0 = acc_v[t, pl.ds(0, lanes)]
                        a1 = acc_v[t, pl.ds(lanes, lanes)]
                        e0 = ego_v[t, pl.ds(0, lanes)]
                        e1 = ego_v[t, pl.ds(lanes, lanes)]
                        out_v[t, pl.ds(0, lanes)] = 0.5 * a0 + coef * e0
                        out_v[t, pl.ds(lanes, lanes)] = 0.5 * a1 + coef * e1
                        return carry2
                    lax.fori_loop(0, ep, comp, 0)

                    o0 = p * qrows + r0

                    @pl.when(c == 0)
                    def _():
                        pltpu.sync_copy(out_v, u_out.at[pl.ds(o0, ep)])

                    @pl.when(c == 1)
                    def _():
                        pltpu.sync_copy(out_v, i_out.at[pl.ds(o0, ep)])
                return carry
            lax.fori_loop(0, nb_per_tile, ep_blk, 0)
            plsc.subcore_barrier()

    return pl.kernel(
        body,
        out_type=[
            jax.ShapeDtypeStruct((n_users, d), jnp.float32),
            jax.ShapeDtypeStruct((n_users, d), jnp.float32),
        ],
        mesh=mesh,
        compiler_params=pltpu.CompilerParams(
            needs_layout_passes=False,
            use_tc_tiling_on_sc=False,
        ),
        scratch_types=[
            pltpu.VMEM((lanes,), jnp.int32),
            pltpu.VMEM((super_,), jnp.int32),
            pltpu.VMEM((super_,), jnp.int32),
            pltpu.VMEM((super_,), jnp.float32),
            pltpu.VMEM((ng, gb), jnp.int32),
            pltpu.VMEM((ng, gb), jnp.int32),
            pltpu.VMEM((super_, d), jnp.float32),
            pltpu.VMEM((ep, d), jnp.float32),
            pltpu.VMEM((ep, d), jnp.float32),
            pltpu.VMEM((ep, d), jnp.float32),
            pltpu.VMEM_SHARED((qrows, d), jnp.float32),
            pltpu.SemaphoreType.DMA,
            pltpu.SemaphoreType.DMA,
        ],
    )


@jax.jit
def _run(user_emb, item_emb, adj_row, adj_col, adj_val):
    ego = jnp.concatenate([user_emb, item_emb], axis=0)
    row = adj_row.astype(jnp.int32)
    col = adj_col.astype(jnp.int32)
    val = adj_val.astype(jnp.float32)
    qrows = N_USERS // _NPASS
    cuts = jnp.arange(1, 2 * _NPASS, dtype=jnp.int32) * qrows
    bs = jnp.searchsorted(row, cuts, side="left").astype(jnp.int32)
    bounds = jnp.zeros((16,), jnp.int32).at[: 2 * _NPASS - 1].set(bs)
    zpad_i = jnp.zeros((_SUPER,), jnp.int32)
    colp = jnp.concatenate([col, zpad_i])
    rowp = jnp.concatenate([row, zpad_i])
    valp = jnp.concatenate([val, jnp.zeros((_SUPER,), jnp.float32)])
    u_g, i_g = _build(N_USERS, NNZ, _SUPER, _GB, _EP, _NPASS)(
        ego, colp, rowp, valp, bounds)
    return (u_g, i_g)


def kernel(user_emb, item_emb, adj_row, adj_col, adj_val):
    return _run(user_emb, item_emb, adj_row, adj_col, adj_val)


# speedup vs baseline: 18.0106x; 4.4600x over previous
import functools

import jax
import jax.numpy as jnp
from jax import lax
from jax.experimental import pallas as pl
from jax.experimental.pallas import tpu as pltpu
from jax.experimental.pallas import tpu_sc as plsc

N_USERS = 50000
N_ITEMS = 50000
N_NODES = N_USERS + N_ITEMS
NNZ = 1600000
D = 32

_SUPER = 1024
_GB = 128
_NG = _SUPER // _GB
_EP = 200
_NPASS = 2


def _build(n_users, nnz, super_, gb, ep, npass):
    d = D
    ng = super_ // gb
    lanes = 16
    n_sub = 16
    qrows = n_users // npass
    nblk = qrows // ep
    nb_per_tile = -(-nblk // n_sub)
    assert qrows % ep == 0 and ep % 8 == 0
    assert super_ % (8 * lanes) == 0 and (ep * d) % lanes == 0

    mesh = plsc.VectorSubcoreMesh(
        core_axis_name="c", subcore_axis_name="s", num_cores=2, num_subcores=n_sub
    )

    def body(ego, colp, rowp, valp, bounds, u_out, i_out,
             bounds_v, colf, rowf, valf, col2d, lidx2d, rows_buf,
             acc_v, ego_v, out_v, accum, sem, sem2):
        c = lax.axis_index("c").astype(jnp.int32)
        s = lax.axis_index("s").astype(jnp.int32)
        i16 = lax.broadcasted_iota(jnp.int32, (lanes,), 0)
        zero16 = jnp.zeros((lanes,), jnp.float32)

        pltpu.sync_copy(bounds, bounds_v)
        bv = bounds_v[...]
        bnd = [jnp.int32(0)] + [bv[q] for q in range(2 * npass - 1)] + [jnp.int32(nnz)]

        coef = jnp.where(c == 0, jnp.float32(0.5), jnp.float32(1.5))

        for p in range(npass):
            lo_sc = jnp.where(c == 0, bnd[p], bnd[npass + p])
            hi_sc = jnp.where(c == 0, bnd[p + 1], bnd[npass + p + 1])
            row_base = c * n_users + p * qrows

            cnt = hi_sc - lo_sc
            per = (cnt + (n_sub - 1)) // n_sub
            lo = lo_sc + jnp.minimum(s * per, cnt)
            hi = lo_sc + jnp.minimum((s + 1) * per, cnt)
            base = (lo // 8) * 8
            n_chunks = (hi - base + (super_ - 1)) // super_

            def zvreg(t, carry):
                acc_v[t, pl.ds(0, lanes)] = zero16
                acc_v[t, pl.ds(lanes, lanes)] = zero16
                return carry
            lax.fori_loop(0, ep, zvreg, 0)

            def zcopy(i, carry):
                bid = s + i * n_sub
                @pl.when(bid < nblk)
                def _():
                    pltpu.sync_copy(acc_v, accum.at[pl.ds(bid * ep, ep)])
                return carry
            lax.fori_loop(0, nb_per_tile, zcopy, 0)
            plsc.subcore_barrier()

            def chunk(g, carry):
                start = base + g * super_
                pltpu.sync_copy(colp.at[pl.ds(start, super_)], colf)
                pltpu.sync_copy(rowp.at[pl.ds(start, super_)], rowf)
                pltpu.sync_copy(valp.at[pl.ds(start, super_)], valf)

                def prep(k, carry2):
                    for sub in range(super_ // lanes // ng):
                        off = sub * lanes
                        el = k * (super_ // ng) + off + i16
                        e = start + el
                        valid = (e >= lo) & (e < hi)
                        cidx = plsc.load_gather(colf, [el])
                        r = plsc.load_gather(rowf, [el])
                        v = plsc.load_gather(valf, [el])
                        lr = jnp.where(valid, r - row_base, 0)
                        v = jnp.where(valid, v, 0.0)
                        plsc.store_scatter(valf, [el], v)
                        col2d[k, pl.ds(off, lanes)] = cidx
                        lidx2d[k, pl.ds(off, lanes)] = lr
                    return carry2
                lax.fori_loop(0, ng, prep, 0)

                gets = [pltpu.async_copy(ego.at[col2d.at[k]],
                                         rows_buf.at[pl.ds(k * gb, gb)], sem)
                        for k in range(ng)]
                for dsc in gets:
                    dsc.wait()

                def scale(j, carry2):
                    off = j * lanes
                    v16 = plsc.load_gather(valf, [off + i16])
                    for ee in range(lanes):
                        e = off + ee
                        v = v16[ee]
                        for h in range(d // lanes):
                            r = rows_buf[e, pl.ds(h * lanes, lanes)]
                            rows_buf[e, pl.ds(h * lanes, lanes)] = r * v
                    return carry2
                lax.fori_loop(0, super_ // lanes, scale, 0)

                puts = [pltpu.async_copy(rows_buf.at[pl.ds(k * gb, gb)],
                                         accum.at[lidx2d.at[k]], sem2, add=True)
                        for k in range(ng)]
                for dsc in puts:
                    dsc.wait()
                return carry
            lax.fori_loop(0, n_chunks, chunk, 0)
            plsc.subcore_barrier()

            def ep_blk(i, carry):
                bid = s + i * n_sub

                @pl.when(bid < nblk)
                def _():
                    r0 = bid * ep
                    pltpu.sync_copy(accum.at[pl.ds(r0, ep)], acc_v)
                    pltpu.sync_copy(ego.at[pl.ds(row_base + r0, ep)], ego_v)

                    def comp(t, carry2):
                        a0 = acc_v[t, pl.ds(0, lanes)]
                        a1 = acc_v[t, pl.ds(lanes, lanes)]
                        e0 = ego_v[t, pl.ds(0, lanes)]
                        e1 = ego_v[t, pl.ds(lanes, lanes)]
                        out_v[t, pl.ds(0, lanes)] = 0.5 * a0 + coef * e0
                        out_v[t, pl.ds(lanes, lanes)] = 0.5 * a1 + coef * e1
                        return carry2
                    lax.fori_loop(0, ep, comp, 0)

                    o0 = p * qrows + r0

                    @pl.when(c == 0)
                    def _():
                        pltpu.sync_copy(out_v, u_out.at[pl.ds(o0, ep)])

                    @pl.when(c == 1)
                    def _():
                        pltpu.sync_copy(out_v, i_out.at[pl.ds(o0, ep)])
                return carry
            lax.fori_loop(0, nb_per_tile, ep_blk, 0)
            plsc.subcore_barrier()

    return pl.kernel(
        body,
        out_type=[
            jax.ShapeDtypeStruct((n_users, d), jnp.float32),
            jax.ShapeDtypeStruct((n_users, d), jnp.float32),
        ],
        mesh=mesh,
        compiler_params=pltpu.CompilerParams(
            needs_layout_passes=False,
            use_tc_tiling_on_sc=False,
        ),
        scratch_types=[
            pltpu.VMEM((lanes,), jnp.int32),
            pltpu.VMEM((super_,), jnp.int32),
            pltpu.VMEM((super_,), jnp.int32),
            pltpu.VMEM((super_,), jnp.float32),
            pltpu.VMEM((ng, gb), jnp.int32),
            pltpu.VMEM((ng, gb), jnp.int32),
            pltpu.VMEM((super_, d), jnp.float32),
            pltpu.VMEM((ep, d), jnp.float32),
            pltpu.VMEM((ep, d), jnp.float32),
            pltpu.VMEM((ep, d), jnp.float32),
            pltpu.VMEM_SHARED((qrows, d), jnp.float32),
            pltpu.SemaphoreType.DMA,
            pltpu.SemaphoreType.DMA,
        ],
    )


@jax.jit
def _run(user_emb, item_emb, adj_row, adj_col, adj_val):
    ego = jnp.concatenate([user_emb, item_emb], axis=0)
    row = adj_row.astype(jnp.int32)
    col = adj_col.astype(jnp.int32)
    val = adj_val.astype(jnp.float32)
    qrows = N_USERS // _NPASS
    cuts = jnp.arange(1, 2 * _NPASS, dtype=jnp.int32) * qrows
    bs = jnp.searchsorted(row, cuts, side="left").astype(jnp.int32)
    bounds = jnp.zeros((16,), jnp.int32).at[: 2 * _NPASS - 1].set(bs)
    zpad_i = jnp.zeros((_SUPER,), jnp.int32)
    colp = jnp.concatenate([col, zpad_i])
    rowp = jnp.concatenate([row, zpad_i])
    valp = jnp.concatenate([val, jnp.zeros((_SUPER,), jnp.float32)])
    u_g, i_g = _build(N_USERS, NNZ, _SUPER, _GB, _EP, _NPASS)(
        ego, colp, rowp, valp, bounds)
    return (u_g, i_g)


def kernel(user_emb, item_emb, adj_row, adj_col, adj_val):
    return _run(user_emb, item_emb, adj_row, adj_col, adj_val)
